# baseline (speedup 1.0000x reference)
import jax
import jax.numpy as jnp
from jax import lax
from jax.experimental import pallas as pl
from jax.experimental.pallas import tpu as pltpu
from jax.experimental.pallas import tpu_sc as plsc

P, B, A, T, S, H = 32, 256, 32, 32, 32, 16384
N = B * A
L = 16
PI = 3.14159265358979

_ACOS_C = (1.5707963050, -0.2145988016, 0.0889789874, -0.0501743046,
           0.0308918810, -0.0170881256, 0.0066700901, -0.0012624911)


def _iota():
    return lax.iota(jnp.int32, L)


def _splat_i(x):
    return jnp.broadcast_to(jnp.asarray(x, jnp.int32), (L,))


def _vrsqrt(x):
    i = lax.bitcast_convert_type(x, jnp.int32)
    y = lax.bitcast_convert_type(jnp.int32(0x5F3759DF) - (i >> 1), jnp.float32)
    half = jnp.float32(0.5) * x
    for _ in range(2):
        y = y * (jnp.float32(1.5) - half * y * y)
    return y


def _vsqrt(x):
    return x * _vrsqrt(x)


def _vacos(c):
    t = jnp.abs(c)
    p = jnp.full((L,), _ACOS_C[7], jnp.float32)
    for a in _ACOS_C[6::-1]:
        p = p * t + jnp.float32(a)
    pos = _vsqrt(jnp.float32(1.0) - t) * p
    return jnp.where(c >= 0, pos, jnp.float32(PI) - pos)


def _cossin_2piv(v):
    a = v * jnp.float32(4.0)
    q = a.astype(jnp.int32)
    z = (a - q.astype(jnp.float32)) * jnp.float32(PI / 2)
    z2 = z * z
    c0 = jnp.float32(1.0) + z2 * (jnp.float32(-0.5) + z2 * (
        jnp.float32(1.0 / 24) + z2 * (jnp.float32(-1.0 / 720)
                                      + z2 * jnp.float32(1.0 / 40320))))
    s0 = z * (jnp.float32(1.0) + z2 * (jnp.float32(-1.0 / 6) + z2 * (
        jnp.float32(1.0 / 120) + z2 * (jnp.float32(-1.0 / 5040)
                                       + z2 * jnp.float32(1.0 / 362880)))))
    q1, q2, q3 = q == 1, q == 2, q == 3
    cos = jnp.where(q1, -s0, jnp.where(q2, -c0, jnp.where(q3, s0, c0)))
    sin = jnp.where(q1, c0, jnp.where(q2, -s0, jnp.where(q3, -c0, s0)))
    return cos, sin


def _gather(ref, idx):
    return plsc.load_gather(ref, [idx])


def _body(cx_h, cy_h, cz_h, bt_h, conns_h, subsT_h, uid_h, wid_h, paths0_h,
          cnt_h, hkeys_h, htab_h, out_h,
          cx_v, cy_v, cz_v, bt_v, conns_v, subsT_v, uid_v, wid_v, paths0_v,
          cnt_v, hk_v, ku_v, hu_v, hw_v, hru_v, hrw_v, pu_v, pw_v, prmT_v, res_v, sem):
    wid = lax.axis_index("s") * 2 + lax.axis_index("c")

    pltpu.sync_copy(cx_h.at[wid], cx_v)
    pltpu.sync_copy(cy_h.at[wid], cy_v)
    pltpu.sync_copy(cz_h.at[wid], cz_v)
    pltpu.sync_copy(bt_h.at[wid], bt_v)
    pltpu.sync_copy(conns_h.at[wid], conns_v)
    pltpu.sync_copy(subsT_h, subsT_v)
    pltpu.sync_copy(uid_h, uid_v)
    pltpu.sync_copy(wid_h, wid_v)
    pltpu.sync_copy(paths0_h, paths0_v)
    pltpu.sync_copy(cnt_h, cnt_v)
    pltpu.sync_copy(hkeys_h, hk_v)

    lanes = _iota()
    eps = jnp.float32(1e-6)

    def keys_body(i, carry):
        ts = i * L + lanes
        t32 = (ts >> 5) * 32
        ku = _splat_i(0)
        kw = _splat_i(0)
        for k, mult in enumerate((131, 31, 7, 1)):
            sub_k = _gather(subsT_v, k * 1024 + ts)
            ku = ku + _gather(uid_v, t32 + sub_k) * mult
            kw = kw + _gather(wid_v, t32 + sub_k) * mult
        plsc.store_scatter(ku_v, [ts], ku)
        hu = ku & (H - 1)
        hw = kw & (H - 1)
        plsc.store_scatter(hu_v, [ts], hu)
        plsc.store_scatter(hw_v, [ts], hw)
        plsc.store_scatter(hru_v, [ts], hu >> 4)
        plsc.store_scatter(hrw_v, [ts], hw >> 4)
        return carry

    lax.fori_loop(0, (T * S) // L, keys_body, 0)

    for j in range(8):
        sl = pl.ds(j * 128, 128)
        cp_u = pltpu.async_copy(htab_h.at[hru_v.at[sl]], pu_v, sem)
        cp_w = pltpu.async_copy(htab_h.at[hrw_v.at[sl]], pw_v, sem)
        cp_u.wait()
        cp_w.wait()

        def prm_body(i, carry):
            ts = j * 128 + i * L + lanes
            t = ts >> 5
            s = ts & 31
            row = (i * L + lanes) & 127
            ku = _gather(ku_v, ts)
            hu = _gather(hu_v, ts)
            hw = _gather(hw_v, ts)
            cu = (hu & 15) * 8
            cw = (hw & 15) * 8
            match = _gather(hk_v, hu) == ku
            prm = []
            for c in range(6):
                pu_c = plsc.load_gather(pu_v, [row, cu + c])
                pw_c = plsc.load_gather(pw_v, [row, cw + c])
                prm.append(jnp.where(match, pu_c, pw_c))
            cp0, sp0 = _cossin_2piv(prm[5])
            cntv = _gather(cnt_v, t)
            maskf = jnp.where(s < cntv, jnp.float32(1.0), jnp.float32(0.0))
            rows = (prm[0], prm[1] * jnp.float32(2.0), prm[2],
                    prm[3] * jnp.float32(PI), prm[4], cp0, sp0, maskf)
            for c, val in enumerate(rows):
                plsc.store_scatter(prmT_v, [c * 1024 + ts], val)
            return carry

        lax.fori_loop(0, 8, prm_body, 0)

    def intra_body(q, acc):
        b = q >> 1
        s0 = (q & 1) * L
        bvec = jnp.broadcast_to(b, (L,))
        tvec = _gather(bt_v, bvec)
        ovec = bvec * 32
        ts = tvec * 32 + s0 + lanes
        xs = []
        for k in range(4):
            sub_k = _gather(subsT_v, k * 1024 + ts)
            gk = ovec + sub_k
            xs.append((_gather(cx_v, gk), _gather(cy_v, gk),
                       _gather(cz_v, gk)))
        x0, x1, x2, x3 = xs
        prm = tuple(_gather(prmT_v, c * 1024 + ts) for c in range(8))
        k_len, l0, k_ang, t0, k_tor, cp0, sp0, maskf = prm

        dx = tuple(x1[c] - x0[c] for c in range(3))
        d01 = _vsqrt(dx[0] * dx[0] + dx[1] * dx[1] + dx[2] * dx[2] + eps)
        uv = tuple(x0[c] - x1[c] for c in range(3))
        vv = tuple(x2[c] - x1[c] for c in range(3))
        s_uv = uv[0] * uv[0] + uv[1] * uv[1] + uv[2] * uv[2] + eps
        s_vv = vv[0] * vv[0] + vv[1] * vv[1] + vv[2] * vv[2] + eps
        dotuv = uv[0] * vv[0] + uv[1] * vv[1] + uv[2] * vv[2]
        cosang = jnp.clip(dotuv * _vrsqrt(s_uv * s_vv),
                          jnp.float32(-1.0 + 1e-6), jnp.float32(1.0 - 1e-6))
        theta = _vacos(cosang)
        b1 = dx
        b2 = vv
        b3 = tuple(x3[c] - x2[c] for c in range(3))

        def cross(u, v):
            return (u[1] * v[2] - u[2] * v[1],
                    u[2] * v[0] - u[0] * v[2],
                    u[0] * v[1] - u[1] * v[0])

        n1 = cross(b1, b2)
        n2 = cross(b2, b3)
        s_b2 = b2[0] * b2[0] + b2[1] * b2[1] + b2[2] * b2[2]
        inv_b2 = jnp.float32(1.0) / (_vsqrt(s_b2) + eps)
        m1 = cross(n1, tuple(b2[c] * inv_b2 for c in range(3)))
        y = m1[0] * n2[0] + m1[1] * n2[1] + m1[2] * n2[2]
        x = n1[0] * n2[0] + n1[1] * n2[1] + n1[2] * n2[2] + eps
        den = x * x + y * y + jnp.float32(1e-30)
        cos2phi = (x * x - y * y) / den
        sin2phi = jnp.float32(2.0) * x * y / den

        dl = d01 - l0
        da = theta - t0
        E = (k_len * dl * dl + k_ang * da * da
             + k_tor * (jnp.float32(1.0) + cos2phi * cp0 + sin2phi * sp0))
        return acc + E * maskf

    acc = lax.fori_loop(0, (B * S) // L, intra_body,
                        jnp.zeros((L,), jnp.float32))

    def inter_body(it, acc):
        e = it * L + lanes
        b = e >> 1
        j = e & 1
        t1 = _gather(bt_v, b)
        ci = b * 4 + j * 2
        b2i = _gather(conns_v, ci)
        c2 = _gather(conns_v, ci + 1) & 1
        t2 = _gather(bt_v, b2i)
        a1 = _gather(paths0_v, t1 * 2 + j)
        a2 = _gather(paths0_v, t2 * 2 + c2)
        g1 = b * 32 + a1
        g2 = b2i * 32 + a2
        d2 = eps
        for cv in (cx_v, cy_v, cz_v):
            dc = _gather(cv, g2) - _gather(cv, g1)
            d2 = d2 + dc * dc
        dd = _vsqrt(d2) - jnp.float32(1.5)
        return acc + jnp.float32(0.5) * dd * dd

    acc = lax.fori_loop(0, (B * 2) // L, inter_body, acc)

    total = jnp.sum(acc)
    res_v[...] = jnp.where(lanes == 0, jnp.broadcast_to(total, (L,)),
                           jnp.float32(0.0))
    pltpu.sync_copy(res_v, out_h.at[wid])


@jax.jit
def _run(cx, cy, cz, bt, conns1, subsT, uidf, widf, paths0, cnts, hkeys,
         htab128):
    mesh = plsc.VectorSubcoreMesh(core_axis_name="c", subcore_axis_name="s")
    f = pl.kernel(
        _body,
        out_type=jax.ShapeDtypeStruct((P, L), jnp.float32),
        mesh=mesh,
        compiler_params=pltpu.CompilerParams(needs_layout_passes=False),
        scratch_types=[
            pltpu.VMEM((N,), jnp.float32),
            pltpu.VMEM((N,), jnp.float32),
            pltpu.VMEM((N,), jnp.float32),
            pltpu.VMEM((B,), jnp.int32),
            pltpu.VMEM((B * 4,), jnp.int32),
            pltpu.VMEM((4 * T * S,), jnp.int32),
            pltpu.VMEM((T * A,), jnp.int32),
            pltpu.VMEM((T * A,), jnp.int32),
            pltpu.VMEM((T * 2,), jnp.int32),
            pltpu.VMEM((T,), jnp.int32),
            pltpu.VMEM((H,), jnp.int32),
            pltpu.VMEM((T * S,), jnp.int32),
            pltpu.VMEM((T * S,), jnp.int32),
            pltpu.VMEM((T * S,), jnp.int32),
            pltpu.VMEM((T * S,), jnp.int32),
            pltpu.VMEM((T * S,), jnp.int32),
            pltpu.VMEM((128, 128), jnp.float32),
            pltpu.VMEM((128, 128), jnp.float32),
            pltpu.VMEM((8 * T * S,), jnp.float32),
            pltpu.VMEM((L,), jnp.float32),
            pltpu.SemaphoreType.DMA,
        ],
    )
    return f(cx, cy, cz, bt, conns1, subsT, uidf, widf, paths0, cnts, hkeys,
             htab128)


def kernel(coords, pose_stack_block_coord_offset, pose_stack_block_types,
           pose_stack_inter_block_connections, atom_paths_from_conn,
           atom_unique_ids, atom_wildcard_ids, hash_keys, hash_values,
           cart_subgraphs, cart_subgraph_offsets, max_subgraphs_per_block):
    cx = coords[:, :, 0]
    cy = coords[:, :, 1]
    cz = coords[:, :, 2]
    conns1 = pose_stack_inter_block_connections.reshape(P, B * 4)
    subsT = cart_subgraphs.transpose(2, 0, 1).reshape(4 * T * S)
    uidf = atom_unique_ids.reshape(T * A)
    widf = atom_wildcard_ids.reshape(T * A)
    paths0 = atom_paths_from_conn[:, :, 0].reshape(T * 2)
    htab128 = jnp.concatenate(
        [hash_values,
         lax.bitcast_convert_type(hash_keys, jnp.float32)[:, None],
         jnp.zeros((H, 1), jnp.float32)], axis=1).reshape(H // 16, 128)
    out = _run(cx, cy, cz, pose_stack_block_types, conns1, subsT, uidf,
               widf, paths0, cart_subgraph_offsets, hash_keys, htab128)
    return out[:, 0]


# speedup vs baseline: 1.4640x; 1.4640x over previous
import jax
import jax.numpy as jnp
from jax import lax
from jax.experimental import pallas as pl
from jax.experimental.pallas import tpu as pltpu
from jax.experimental.pallas import tpu_sc as plsc

P, B, A, T, S, H = 32, 256, 32, 32, 32, 16384
N = B * A
L = 16
PI = 3.14159265358979

_ACOS_C = (1.5707963050, -0.2145988016, 0.0889789874, -0.0501743046,
           0.0308918810, -0.0170881256, 0.0066700901, -0.0012624911)


def _iota():
    return lax.iota(jnp.int32, L)


def _splat_i(x):
    return jnp.broadcast_to(jnp.asarray(x, jnp.int32), (L,))


def _vrsqrt(x):
    i = lax.bitcast_convert_type(x, jnp.int32)
    y = lax.bitcast_convert_type(jnp.int32(0x5F3759DF) - (i >> 1), jnp.float32)
    half = jnp.float32(0.5) * x
    for _ in range(2):
        y = y * (jnp.float32(1.5) - half * y * y)
    return y


def _vsqrt(x):
    return x * _vrsqrt(x)


def _vacos(c):
    t = jnp.abs(c)
    p = jnp.full((L,), _ACOS_C[7], jnp.float32)
    for a in _ACOS_C[6::-1]:
        p = p * t + jnp.float32(a)
    pos = _vsqrt(jnp.float32(1.0) - t) * p
    return jnp.where(c >= 0, pos, jnp.float32(PI) - pos)


def _cossin_2piv(v):
    a = v * jnp.float32(4.0)
    q = a.astype(jnp.int32)
    z = (a - q.astype(jnp.float32)) * jnp.float32(PI / 2)
    z2 = z * z
    c0 = jnp.float32(1.0) + z2 * (jnp.float32(-0.5) + z2 * (
        jnp.float32(1.0 / 24) + z2 * (jnp.float32(-1.0 / 720)
                                      + z2 * jnp.float32(1.0 / 40320))))
    s0 = z * (jnp.float32(1.0) + z2 * (jnp.float32(-1.0 / 6) + z2 * (
        jnp.float32(1.0 / 120) + z2 * (jnp.float32(-1.0 / 5040)
                                       + z2 * jnp.float32(1.0 / 362880)))))
    q1, q2, q3 = q == 1, q == 2, q == 3
    cos = jnp.where(q1, -s0, jnp.where(q2, -c0, jnp.where(q3, s0, c0)))
    sin = jnp.where(q1, c0, jnp.where(q2, -s0, jnp.where(q3, -c0, s0)))
    return cos, sin


def _gather(ref, idx):
    return plsc.load_gather(ref, [idx])


def _body(cx_h, cy_h, cz_h, bt_h, conns_h, subsT_h, uid_h, wid_h, paths0_h,
          cnt_h, hkeys_h, htab_h, out_h,
          cx_v, cy_v, cz_v, bt_v, conns_v, subsT_v, uid_v, wid_v, paths0_v,
          cnt_v, hk_v, ku_v, hu_v, hw_v, hru_v, hrw_v, pu_v, pw_v, prml_v, shared_v, prmT_v, res_v, sem):
    sid = lax.axis_index("s")
    wid = sid * 2 + lax.axis_index("c")

    pltpu.sync_copy(cx_h.at[wid], cx_v)
    pltpu.sync_copy(cy_h.at[wid], cy_v)
    pltpu.sync_copy(cz_h.at[wid], cz_v)
    pltpu.sync_copy(bt_h.at[wid], bt_v)
    pltpu.sync_copy(conns_h.at[wid], conns_v)
    pltpu.sync_copy(subsT_h, subsT_v)
    pltpu.sync_copy(uid_h, uid_v)
    pltpu.sync_copy(wid_h, wid_v)
    pltpu.sync_copy(paths0_h, paths0_v)
    pltpu.sync_copy(cnt_h, cnt_v)
    pltpu.sync_copy(hkeys_h, hk_v)

    lanes = _iota()
    eps = jnp.float32(1e-6)

    ebase = sid * 64

    def keys_body(i, carry):
        loc = i * L + lanes
        ts = ebase + loc
        t32 = (ts >> 5) * 32
        ku = _splat_i(0)
        kw = _splat_i(0)
        for k, mult in enumerate((131, 31, 7, 1)):
            sub_k = _gather(subsT_v, k * 1024 + ts)
            ku = ku + _gather(uid_v, t32 + sub_k) * mult
            kw = kw + _gather(wid_v, t32 + sub_k) * mult
        plsc.store_scatter(ku_v, [loc], ku)
        hu = ku & (H - 1)
        hw = kw & (H - 1)
        plsc.store_scatter(hu_v, [loc], hu)
        plsc.store_scatter(hw_v, [loc], hw)
        plsc.store_scatter(hru_v, [loc], hu >> 4)
        plsc.store_scatter(hrw_v, [loc], hw >> 4)
        return carry

    lax.fori_loop(0, 64 // L, keys_body, 0)

    cp_u = pltpu.async_copy(htab_h.at[hru_v], pu_v, sem)
    cp_w = pltpu.async_copy(htab_h.at[hrw_v], pw_v, sem)
    cp_u.wait()
    cp_w.wait()

    def prm_body(i, carry):
        loc = i * L + lanes
        ts = ebase + loc
        t = ts >> 5
        s = ts & 31
        ku = _gather(ku_v, loc)
        hu = _gather(hu_v, loc)
        hw = _gather(hw_v, loc)
        cu = (hu & 15) * 8
        cw = (hw & 15) * 8
        match = _gather(hk_v, hu) == ku
        prm = []
        for c in range(6):
            pu_c = plsc.load_gather(pu_v, [loc, cu + c])
            pw_c = plsc.load_gather(pw_v, [loc, cw + c])
            prm.append(jnp.where(match, pu_c, pw_c))
        cp0, sp0 = _cossin_2piv(prm[5])
        cntv = _gather(cnt_v, t)
        maskf = jnp.where(s < cntv, jnp.float32(1.0), jnp.float32(0.0))
        rows = (prm[0], prm[1] * jnp.float32(2.0), prm[2],
                prm[3] * jnp.float32(PI), prm[4], cp0, sp0, maskf)
        for c, val in enumerate(rows):
            plsc.store_scatter(prml_v, [loc * 8 + c], val)
        return carry

    lax.fori_loop(0, 64 // L, prm_body, 0)

    pltpu.sync_copy(prml_v, shared_v.at[pl.ds(sid * 512, 512)])
    plsc.subcore_barrier()
    pltpu.sync_copy(shared_v, prmT_v)

    def intra_body(q, acc):
        b = q >> 1
        s0 = (q & 1) * L
        bvec = jnp.broadcast_to(b, (L,))
        tvec = _gather(bt_v, bvec)
        ovec = bvec * 32
        ts = tvec * 32 + s0 + lanes
        xs = []
        for k in range(4):
            sub_k = _gather(subsT_v, k * 1024 + ts)
            gk = ovec + sub_k
            xs.append((_gather(cx_v, gk), _gather(cy_v, gk),
                       _gather(cz_v, gk)))
        x0, x1, x2, x3 = xs
        prm = tuple(_gather(prmT_v, ts * 8 + c) for c in range(8))
        k_len, l0, k_ang, t0, k_tor, cp0, sp0, maskf = prm

        dx = tuple(x1[c] - x0[c] for c in range(3))
        d01 = _vsqrt(dx[0] * dx[0] + dx[1] * dx[1] + dx[2] * dx[2] + eps)
        uv = tuple(x0[c] - x1[c] for c in range(3))
        vv = tuple(x2[c] - x1[c] for c in range(3))
        s_uv = uv[0] * uv[0] + uv[1] * uv[1] + uv[2] * uv[2] + eps
        s_vv = vv[0] * vv[0] + vv[1] * vv[1] + vv[2] * vv[2] + eps
        dotuv = uv[0] * vv[0] + uv[1] * vv[1] + uv[2] * vv[2]
        cosang = jnp.clip(dotuv * _vrsqrt(s_uv * s_vv),
                          jnp.float32(-1.0 + 1e-6), jnp.float32(1.0 - 1e-6))
        theta = _vacos(cosang)
        b1 = dx
        b2 = vv
        b3 = tuple(x3[c] - x2[c] for c in range(3))

        def cross(u, v):
            return (u[1] * v[2] - u[2] * v[1],
                    u[2] * v[0] - u[0] * v[2],
                    u[0] * v[1] - u[1] * v[0])

        n1 = cross(b1, b2)
        n2 = cross(b2, b3)
        s_b2 = b2[0] * b2[0] + b2[1] * b2[1] + b2[2] * b2[2]
        inv_b2 = jnp.float32(1.0) / (_vsqrt(s_b2) + eps)
        m1 = cross(n1, tuple(b2[c] * inv_b2 for c in range(3)))
        y = m1[0] * n2[0] + m1[1] * n2[1] + m1[2] * n2[2]
        x = n1[0] * n2[0] + n1[1] * n2[1] + n1[2] * n2[2] + eps
        den = x * x + y * y + jnp.float32(1e-30)
        cos2phi = (x * x - y * y) / den
        sin2phi = jnp.float32(2.0) * x * y / den

        dl = d01 - l0
        da = theta - t0
        E = (k_len * dl * dl + k_ang * da * da
             + k_tor * (jnp.float32(1.0) + cos2phi * cp0 + sin2phi * sp0))
        return acc + E * maskf

    acc = lax.fori_loop(0, (B * S) // L, intra_body,
                        jnp.zeros((L,), jnp.float32))

    def inter_body(it, acc):
        e = it * L + lanes
        b = e >> 1
        j = e & 1
        t1 = _gather(bt_v, b)
        ci = b * 4 + j * 2
        b2i = _gather(conns_v, ci)
        c2 = _gather(conns_v, ci + 1) & 1
        t2 = _gather(bt_v, b2i)
        a1 = _gather(paths0_v, t1 * 2 + j)
        a2 = _gather(paths0_v, t2 * 2 + c2)
        g1 = b * 32 + a1
        g2 = b2i * 32 + a2
        d2 = eps
        for cv in (cx_v, cy_v, cz_v):
            dc = _gather(cv, g2) - _gather(cv, g1)
            d2 = d2 + dc * dc
        dd = _vsqrt(d2) - jnp.float32(1.5)
        return acc + jnp.float32(0.5) * dd * dd

    acc = lax.fori_loop(0, (B * 2) // L, inter_body, acc)

    total = jnp.sum(acc)
    res_v[...] = jnp.where(lanes == 0, jnp.broadcast_to(total, (L,)),
                           jnp.float32(0.0))
    pltpu.sync_copy(res_v, out_h.at[wid])


@jax.jit
def _run(cx, cy, cz, bt, conns1, subsT, uidf, widf, paths0, cnts, hkeys,
         htab128):
    mesh = plsc.VectorSubcoreMesh(core_axis_name="c", subcore_axis_name="s")
    f = pl.kernel(
        _body,
        out_type=jax.ShapeDtypeStruct((P, L), jnp.float32),
        mesh=mesh,
        compiler_params=pltpu.CompilerParams(needs_layout_passes=False),
        scratch_types=[
            pltpu.VMEM((N,), jnp.float32),
            pltpu.VMEM((N,), jnp.float32),
            pltpu.VMEM((N,), jnp.float32),
            pltpu.VMEM((B,), jnp.int32),
            pltpu.VMEM((B * 4,), jnp.int32),
            pltpu.VMEM((4 * T * S,), jnp.int32),
            pltpu.VMEM((T * A,), jnp.int32),
            pltpu.VMEM((T * A,), jnp.int32),
            pltpu.VMEM((T * 2,), jnp.int32),
            pltpu.VMEM((T,), jnp.int32),
            pltpu.VMEM((H,), jnp.int32),
            pltpu.VMEM((64,), jnp.int32),
            pltpu.VMEM((64,), jnp.int32),
            pltpu.VMEM((64,), jnp.int32),
            pltpu.VMEM((64,), jnp.int32),
            pltpu.VMEM((64,), jnp.int32),
            pltpu.VMEM((64, 128), jnp.float32),
            pltpu.VMEM((64, 128), jnp.float32),
            pltpu.VMEM((512,), jnp.float32),
            pltpu.VMEM_SHARED((T * S * 8,), jnp.float32),
            pltpu.VMEM((T * S * 8,), jnp.float32),
            pltpu.VMEM((L,), jnp.float32),
            pltpu.SemaphoreType.DMA,
        ],
    )
    return f(cx, cy, cz, bt, conns1, subsT, uidf, widf, paths0, cnts, hkeys,
             htab128)


def kernel(coords, pose_stack_block_coord_offset, pose_stack_block_types,
           pose_stack_inter_block_connections, atom_paths_from_conn,
           atom_unique_ids, atom_wildcard_ids, hash_keys, hash_values,
           cart_subgraphs, cart_subgraph_offsets, max_subgraphs_per_block):
    cx = coords[:, :, 0]
    cy = coords[:, :, 1]
    cz = coords[:, :, 2]
    conns1 = pose_stack_inter_block_connections.reshape(P, B * 4)
    subsT = cart_subgraphs.transpose(2, 0, 1).reshape(4 * T * S)
    uidf = atom_unique_ids.reshape(T * A)
    widf = atom_wildcard_ids.reshape(T * A)
    paths0 = atom_paths_from_conn[:, :, 0].reshape(T * 2)
    htab128 = jnp.concatenate(
        [hash_values,
         lax.bitcast_convert_type(hash_keys, jnp.float32)[:, None],
         jnp.zeros((H, 1), jnp.float32)], axis=1).reshape(H // 16, 128)
    out = _run(cx, cy, cz, pose_stack_block_types, conns1, subsT, uidf,
               widf, paths0, cart_subgraph_offsets, hash_keys, htab128)
    return out[:, 0]


# speedup vs baseline: 1.5755x; 1.0762x over previous
import jax
import jax.numpy as jnp
from jax import lax
from jax.experimental import pallas as pl
from jax.experimental.pallas import tpu as pltpu
from jax.experimental.pallas import tpu_sc as plsc

P, B, A, T, S, H = 32, 256, 32, 32, 32, 16384
N = B * A
L = 16
PI = 3.14159265358979

_ACOS_C = (1.5707963050, -0.2145988016, 0.0889789874, -0.0501743046,
           0.0308918810, -0.0170881256, 0.0066700901, -0.0012624911)


def _iota():
    return lax.iota(jnp.int32, L)


def _splat_i(x):
    return jnp.broadcast_to(jnp.asarray(x, jnp.int32), (L,))


def _vrsqrt(x):
    i = lax.bitcast_convert_type(x, jnp.int32)
    y = lax.bitcast_convert_type(jnp.int32(0x5F3759DF) - (i >> 1), jnp.float32)
    half = jnp.float32(0.5) * x
    for _ in range(2):
        y = y * (jnp.float32(1.5) - half * y * y)
    return y


def _vsqrt(x):
    return x * _vrsqrt(x)


def _vacos(c):
    t = jnp.abs(c)
    p = jnp.full((L,), _ACOS_C[7], jnp.float32)
    for a in _ACOS_C[6::-1]:
        p = p * t + jnp.float32(a)
    pos = _vsqrt(jnp.float32(1.0) - t) * p
    return jnp.where(c >= 0, pos, jnp.float32(PI) - pos)


def _cossin_2piv(v):
    a = v * jnp.float32(4.0)
    q = a.astype(jnp.int32)
    z = (a - q.astype(jnp.float32)) * jnp.float32(PI / 2)
    z2 = z * z
    c0 = jnp.float32(1.0) + z2 * (jnp.float32(-0.5) + z2 * (
        jnp.float32(1.0 / 24) + z2 * (jnp.float32(-1.0 / 720)
                                      + z2 * jnp.float32(1.0 / 40320))))
    s0 = z * (jnp.float32(1.0) + z2 * (jnp.float32(-1.0 / 6) + z2 * (
        jnp.float32(1.0 / 120) + z2 * (jnp.float32(-1.0 / 5040)
                                       + z2 * jnp.float32(1.0 / 362880)))))
    q1, q2, q3 = q == 1, q == 2, q == 3
    cos = jnp.where(q1, -s0, jnp.where(q2, -c0, jnp.where(q3, s0, c0)))
    sin = jnp.where(q1, c0, jnp.where(q2, -s0, jnp.where(q3, -c0, s0)))
    return cos, sin


def _gather(ref, idx):
    return plsc.load_gather(ref, [idx])


def _body(cx_h, cy_h, cz_h, bt_h, conns_h, subsT_h, uid_h, wid_h, paths0_h,
          cnt_h, hkeys_h, hv0_h, hv1_h, hv2_h, hv3_h, hv4_h, hv5_h, out_h,
          cx_v, cy_v, cz_v, bt_v, conns_v, subsT_v, uid_v, wid_v, paths0_v,
          cnt_v, hk_v, ku_v, hu_v, hw_v,
          pu0_v, pu1_v, pu2_v, pu3_v, pu4_v, pu5_v,
          pw0_v, pw1_v, pw2_v, pw3_v, pw4_v, pw5_v,
          prml_v, shared_v, prmT_v, res_v, sem):
    sid = lax.axis_index("s")
    wid = sid * 2 + lax.axis_index("c")

    pltpu.sync_copy(cx_h.at[wid], cx_v)
    pltpu.sync_copy(cy_h.at[wid], cy_v)
    pltpu.sync_copy(cz_h.at[wid], cz_v)
    pltpu.sync_copy(bt_h.at[wid], bt_v)
    pltpu.sync_copy(conns_h.at[wid], conns_v)
    pltpu.sync_copy(subsT_h, subsT_v)
    pltpu.sync_copy(uid_h, uid_v)
    pltpu.sync_copy(wid_h, wid_v)
    pltpu.sync_copy(paths0_h, paths0_v)
    pltpu.sync_copy(cnt_h, cnt_v)
    pltpu.sync_copy(hkeys_h, hk_v)

    lanes = _iota()
    eps = jnp.float32(1e-6)

    ebase = sid * 64

    def keys_body(i, carry):
        loc = i * L + lanes
        ts = ebase + loc
        t32 = (ts >> 5) * 32
        ku = _splat_i(0)
        kw = _splat_i(0)
        for k, mult in enumerate((131, 31, 7, 1)):
            sub_k = _gather(subsT_v, k * 1024 + ts)
            ku = ku + _gather(uid_v, t32 + sub_k) * mult
            kw = kw + _gather(wid_v, t32 + sub_k) * mult
        plsc.store_scatter(ku_v, [loc], ku)
        hu = ku & (H - 1)
        hw = kw & (H - 1)
        plsc.store_scatter(hu_v, [loc], hu)
        plsc.store_scatter(hw_v, [loc], hw)
        return carry

    lax.fori_loop(0, 64 // L, keys_body, 0)

    hv_hs = (hv0_h, hv1_h, hv2_h, hv3_h, hv4_h, hv5_h)
    pu_vs = (pu0_v, pu1_v, pu2_v, pu3_v, pu4_v, pu5_v)
    pw_vs = (pw0_v, pw1_v, pw2_v, pw3_v, pw4_v, pw5_v)
    copies = []
    for c in range(6):
        copies.append(pltpu.async_copy(hv_hs[c].at[hu_v], pu_vs[c], sem))
        copies.append(pltpu.async_copy(hv_hs[c].at[hw_v], pw_vs[c], sem))
    for cp in copies:
        cp.wait()

    def prm_body(i, carry):
        loc = i * L + lanes
        ts = ebase + loc
        t = ts >> 5
        s = ts & 31
        ku = _gather(ku_v, loc)
        hu = _gather(hu_v, loc)
        match = _gather(hk_v, hu) == ku
        prm = []
        for c in range(6):
            pu_c = _gather(pu_vs[c], loc)
            pw_c = _gather(pw_vs[c], loc)
            prm.append(jnp.where(match, pu_c, pw_c))
        cp0, sp0 = _cossin_2piv(prm[5])
        cntv = _gather(cnt_v, t)
        maskf = jnp.where(s < cntv, jnp.float32(1.0), jnp.float32(0.0))
        rows = (prm[0], prm[1] * jnp.float32(2.0), prm[2],
                prm[3] * jnp.float32(PI), prm[4], cp0, sp0, maskf)
        for c, val in enumerate(rows):
            plsc.store_scatter(prml_v, [loc * 8 + c], val)
        return carry

    lax.fori_loop(0, 64 // L, prm_body, 0)

    pltpu.sync_copy(prml_v, shared_v.at[pl.ds(sid * 512, 512)])
    plsc.subcore_barrier()
    pltpu.sync_copy(shared_v, prmT_v)

    def intra_body(q, acc):
        b = q >> 1
        s0 = (q & 1) * L
        bvec = jnp.broadcast_to(b, (L,))
        tvec = _gather(bt_v, bvec)
        ovec = bvec * 32
        ts = tvec * 32 + s0 + lanes
        xs = []
        for k in range(4):
            sub_k = _gather(subsT_v, k * 1024 + ts)
            gk = ovec + sub_k
            xs.append((_gather(cx_v, gk), _gather(cy_v, gk),
                       _gather(cz_v, gk)))
        x0, x1, x2, x3 = xs
        prm = tuple(_gather(prmT_v, ts * 8 + c) for c in range(8))
        k_len, l0, k_ang, t0, k_tor, cp0, sp0, maskf = prm

        dx = tuple(x1[c] - x0[c] for c in range(3))
        d01 = _vsqrt(dx[0] * dx[0] + dx[1] * dx[1] + dx[2] * dx[2] + eps)
        uv = tuple(x0[c] - x1[c] for c in range(3))
        vv = tuple(x2[c] - x1[c] for c in range(3))
        s_uv = uv[0] * uv[0] + uv[1] * uv[1] + uv[2] * uv[2] + eps
        s_vv = vv[0] * vv[0] + vv[1] * vv[1] + vv[2] * vv[2] + eps
        dotuv = uv[0] * vv[0] + uv[1] * vv[1] + uv[2] * vv[2]
        cosang = jnp.clip(dotuv * _vrsqrt(s_uv * s_vv),
                          jnp.float32(-1.0 + 1e-6), jnp.float32(1.0 - 1e-6))
        theta = _vacos(cosang)
        b1 = dx
        b2 = vv
        b3 = tuple(x3[c] - x2[c] for c in range(3))

        def cross(u, v):
            return (u[1] * v[2] - u[2] * v[1],
                    u[2] * v[0] - u[0] * v[2],
                    u[0] * v[1] - u[1] * v[0])

        n1 = cross(b1, b2)
        n2 = cross(b2, b3)
        s_b2 = b2[0] * b2[0] + b2[1] * b2[1] + b2[2] * b2[2]
        inv_b2 = jnp.float32(1.0) / (_vsqrt(s_b2) + eps)
        m1 = cross(n1, tuple(b2[c] * inv_b2 for c in range(3)))
        y = m1[0] * n2[0] + m1[1] * n2[1] + m1[2] * n2[2]
        x = n1[0] * n2[0] + n1[1] * n2[1] + n1[2] * n2[2] + eps
        den = x * x + y * y + jnp.float32(1e-30)
        cos2phi = (x * x - y * y) / den
        sin2phi = jnp.float32(2.0) * x * y / den

        dl = d01 - l0
        da = theta - t0
        E = (k_len * dl * dl + k_ang * da * da
             + k_tor * (jnp.float32(1.0) + cos2phi * cp0 + sin2phi * sp0))
        return acc + E * maskf

    acc = lax.fori_loop(0, (B * S) // L, intra_body,
                        jnp.zeros((L,), jnp.float32))

    def inter_body(it, acc):
        e = it * L + lanes
        b = e >> 1
        j = e & 1
        t1 = _gather(bt_v, b)
        ci = b * 4 + j * 2
        b2i = _gather(conns_v, ci)
        c2 = _gather(conns_v, ci + 1) & 1
        t2 = _gather(bt_v, b2i)
        a1 = _gather(paths0_v, t1 * 2 + j)
        a2 = _gather(paths0_v, t2 * 2 + c2)
        g1 = b * 32 + a1
        g2 = b2i * 32 + a2
        d2 = eps
        for cv in (cx_v, cy_v, cz_v):
            dc = _gather(cv, g2) - _gather(cv, g1)
            d2 = d2 + dc * dc
        dd = _vsqrt(d2) - jnp.float32(1.5)
        return acc + jnp.float32(0.5) * dd * dd

    acc = lax.fori_loop(0, (B * 2) // L, inter_body, acc)

    total = jnp.sum(acc)
    res_v[...] = jnp.where(lanes == 0, jnp.broadcast_to(total, (L,)),
                           jnp.float32(0.0))
    pltpu.sync_copy(res_v, out_h.at[wid])


@jax.jit
def _run(cx, cy, cz, bt, conns1, subsT, uidf, widf, paths0, cnts, hkeys,
         hv0, hv1, hv2, hv3, hv4, hv5):
    mesh = plsc.VectorSubcoreMesh(core_axis_name="c", subcore_axis_name="s")
    f = pl.kernel(
        _body,
        out_type=jax.ShapeDtypeStruct((P, L), jnp.float32),
        mesh=mesh,
        compiler_params=pltpu.CompilerParams(needs_layout_passes=False),
        scratch_types=[
            pltpu.VMEM((N,), jnp.float32),
            pltpu.VMEM((N,), jnp.float32),
            pltpu.VMEM((N,), jnp.float32),
            pltpu.VMEM((B,), jnp.int32),
            pltpu.VMEM((B * 4,), jnp.int32),
            pltpu.VMEM((4 * T * S,), jnp.int32),
            pltpu.VMEM((T * A,), jnp.int32),
            pltpu.VMEM((T * A,), jnp.int32),
            pltpu.VMEM((T * 2,), jnp.int32),
            pltpu.VMEM((T,), jnp.int32),
            pltpu.VMEM((H,), jnp.int32),
            pltpu.VMEM((64,), jnp.int32),
            pltpu.VMEM((64,), jnp.int32),
            pltpu.VMEM((64,), jnp.int32),
            pltpu.VMEM((64,), jnp.float32),
            pltpu.VMEM((64,), jnp.float32),
            pltpu.VMEM((64,), jnp.float32),
            pltpu.VMEM((64,), jnp.float32),
            pltpu.VMEM((64,), jnp.float32),
            pltpu.VMEM((64,), jnp.float32),
            pltpu.VMEM((64,), jnp.float32),
            pltpu.VMEM((64,), jnp.float32),
            pltpu.VMEM((64,), jnp.float32),
            pltpu.VMEM((64,), jnp.float32),
            pltpu.VMEM((64,), jnp.float32),
            pltpu.VMEM((64,), jnp.float32),
            pltpu.VMEM((512,), jnp.float32),
            pltpu.VMEM_SHARED((T * S * 8,), jnp.float32),
            pltpu.VMEM((T * S * 8,), jnp.float32),
            pltpu.VMEM((L,), jnp.float32),
            pltpu.SemaphoreType.DMA,
        ],
    )
    return f(cx, cy, cz, bt, conns1, subsT, uidf, widf, paths0, cnts, hkeys,
             hv0, hv1, hv2, hv3, hv4, hv5)


def kernel(coords, pose_stack_block_coord_offset, pose_stack_block_types,
           pose_stack_inter_block_connections, atom_paths_from_conn,
           atom_unique_ids, atom_wildcard_ids, hash_keys, hash_values,
           cart_subgraphs, cart_subgraph_offsets, max_subgraphs_per_block):
    cx = coords[:, :, 0]
    cy = coords[:, :, 1]
    cz = coords[:, :, 2]
    conns1 = pose_stack_inter_block_connections.reshape(P, B * 4)
    subsT = cart_subgraphs.transpose(2, 0, 1).reshape(4 * T * S)
    uidf = atom_unique_ids.reshape(T * A)
    widf = atom_wildcard_ids.reshape(T * A)
    paths0 = atom_paths_from_conn[:, :, 0].reshape(T * 2)
    hv = [hash_values[:, c] for c in range(6)]
    out = _run(cx, cy, cz, pose_stack_block_types, conns1, subsT, uidf,
               widf, paths0, cart_subgraph_offsets, hash_keys, *hv)
    return out[:, 0]


# speedup vs baseline: 1.6739x; 1.0625x over previous
import jax
import jax.numpy as jnp
from jax import lax
from jax.experimental import pallas as pl
from jax.experimental.pallas import tpu as pltpu
from jax.experimental.pallas import tpu_sc as plsc

P, B, A, T, S, H = 32, 256, 32, 32, 32, 16384
N = B * A
L = 16
PI = 3.14159265358979

_ACOS_C = (1.5707963050, -0.2145988016, 0.0889789874, -0.0501743046,
           0.0308918810, -0.0170881256, 0.0066700901, -0.0012624911)


def _iota():
    return lax.iota(jnp.int32, L)


def _splat_i(x):
    return jnp.broadcast_to(jnp.asarray(x, jnp.int32), (L,))


def _vrsqrt(x):
    i = lax.bitcast_convert_type(x, jnp.int32)
    y = lax.bitcast_convert_type(jnp.int32(0x5F3759DF) - (i >> 1), jnp.float32)
    half = jnp.float32(0.5) * x
    for _ in range(2):
        y = y * (jnp.float32(1.5) - half * y * y)
    return y


def _vsqrt(x):
    return x * _vrsqrt(x)


def _vacos(c):
    t = jnp.abs(c)
    p = jnp.full((L,), _ACOS_C[7], jnp.float32)
    for a in _ACOS_C[6::-1]:
        p = p * t + jnp.float32(a)
    pos = _vsqrt(jnp.float32(1.0) - t) * p
    return jnp.where(c >= 0, pos, jnp.float32(PI) - pos)


def _cossin_2piv(v):
    a = v * jnp.float32(4.0)
    q = a.astype(jnp.int32)
    z = (a - q.astype(jnp.float32)) * jnp.float32(PI / 2)
    z2 = z * z
    c0 = jnp.float32(1.0) + z2 * (jnp.float32(-0.5) + z2 * (
        jnp.float32(1.0 / 24) + z2 * (jnp.float32(-1.0 / 720)
                                      + z2 * jnp.float32(1.0 / 40320))))
    s0 = z * (jnp.float32(1.0) + z2 * (jnp.float32(-1.0 / 6) + z2 * (
        jnp.float32(1.0 / 120) + z2 * (jnp.float32(-1.0 / 5040)
                                       + z2 * jnp.float32(1.0 / 362880)))))
    q1, q2, q3 = q == 1, q == 2, q == 3
    cos = jnp.where(q1, -s0, jnp.where(q2, -c0, jnp.where(q3, s0, c0)))
    sin = jnp.where(q1, c0, jnp.where(q2, -s0, jnp.where(q3, -c0, s0)))
    return cos, sin


def _gather(ref, idx):
    return plsc.load_gather(ref, [idx])


def _body(cx_h, cy_h, cz_h, bt_h, conns_h, tab_h,
          hv0_h, hv1_h, hv2_h, hv3_h, hv4_h, hv5_h, out_h,
          cx_v, cy_v, cz_v, bt_v, conns_v, tab_v, ku_v, hu_v, hw_v,
          pu0_v, pu1_v, pu2_v, pu3_v, pu4_v, pu5_v,
          pw0_v, pw1_v, pw2_v, pw3_v, pw4_v, pw5_v,
          prml_v, shared_v, prmT_v, res_v, sem):
    sid = lax.axis_index("s")
    wid = sid * 2 + lax.axis_index("c")

    pltpu.sync_copy(cx_h.at[wid], cx_v)
    pltpu.sync_copy(cy_h.at[wid], cy_v)
    pltpu.sync_copy(cz_h.at[wid], cz_v)
    pltpu.sync_copy(bt_h.at[wid], bt_v)
    pltpu.sync_copy(conns_h.at[wid], conns_v)
    pltpu.sync_copy(tab_h, tab_v)

    lanes = _iota()
    eps = jnp.float32(1e-6)

    ebase = sid * 64

    def keys_body(i, carry):
        loc = i * L + lanes
        ts = ebase + loc
        t32 = (ts >> 5) * 32
        ku = _splat_i(0)
        kw = _splat_i(0)
        for k, mult in enumerate((131, 31, 7, 1)):
            sub_k = _gather(tab_v, k * 1024 + ts)
            ku = ku + _gather(tab_v, 4096 + t32 + sub_k) * mult
            kw = kw + _gather(tab_v, 5120 + t32 + sub_k) * mult
        plsc.store_scatter(ku_v, [loc], ku)
        hu = ku & (H - 1)
        hw = kw & (H - 1)
        plsc.store_scatter(hu_v, [loc], hu)
        plsc.store_scatter(hw_v, [loc], hw)
        return carry

    lax.fori_loop(0, 64 // L, keys_body, 0)

    hv_hs = (hv0_h, hv1_h, hv2_h, hv3_h, hv4_h, hv5_h)
    pu_vs = (pu0_v, pu1_v, pu2_v, pu3_v, pu4_v, pu5_v)
    pw_vs = (pw0_v, pw1_v, pw2_v, pw3_v, pw4_v, pw5_v)
    copies = []
    for c in range(6):
        copies.append(pltpu.async_copy(hv_hs[c].at[hu_v], pu_vs[c], sem))
        copies.append(pltpu.async_copy(hv_hs[c].at[hw_v], pw_vs[c], sem))
    for cp in copies:
        cp.wait()

    def prm_body(i, carry):
        loc = i * L + lanes
        ts = ebase + loc
        t = ts >> 5
        s = ts & 31
        ku = _gather(ku_v, loc)
        hu = _gather(hu_v, loc)
        match = _gather(tab_v, 6240 + hu) == ku
        prm = []
        for c in range(6):
            pu_c = _gather(pu_vs[c], loc)
            pw_c = _gather(pw_vs[c], loc)
            prm.append(jnp.where(match, pu_c, pw_c))
        cp0, sp0 = _cossin_2piv(prm[5])
        cntv = _gather(tab_v, 6208 + t)
        maskf = jnp.where(s < cntv, jnp.float32(1.0), jnp.float32(0.0))
        rows = (prm[0], prm[1] * jnp.float32(2.0), prm[2],
                prm[3] * jnp.float32(PI), prm[4], cp0, sp0, maskf)
        for c, val in enumerate(rows):
            plsc.store_scatter(prml_v, [loc * 8 + c], val)
        return carry

    lax.fori_loop(0, 64 // L, prm_body, 0)

    pltpu.sync_copy(prml_v, shared_v.at[pl.ds(sid * 512, 512)])
    plsc.subcore_barrier()
    pltpu.sync_copy(shared_v, prmT_v)

    def intra_body(q, acc):
        b = q >> 1
        s0 = (q & 1) * L
        bvec = jnp.broadcast_to(b, (L,))
        tvec = _gather(bt_v, bvec)
        ovec = bvec * 32
        ts = tvec * 32 + s0 + lanes
        xs = []
        for k in range(4):
            sub_k = _gather(tab_v, k * 1024 + ts)
            gk = ovec + sub_k
            xs.append((_gather(cx_v, gk), _gather(cy_v, gk),
                       _gather(cz_v, gk)))
        x0, x1, x2, x3 = xs
        prm = tuple(_gather(prmT_v, ts * 8 + c) for c in range(8))
        k_len, l0, k_ang, t0, k_tor, cp0, sp0, maskf = prm

        dx = tuple(x1[c] - x0[c] for c in range(3))
        d01 = _vsqrt(dx[0] * dx[0] + dx[1] * dx[1] + dx[2] * dx[2] + eps)
        uv = tuple(x0[c] - x1[c] for c in range(3))
        vv = tuple(x2[c] - x1[c] for c in range(3))
        s_uv = uv[0] * uv[0] + uv[1] * uv[1] + uv[2] * uv[2] + eps
        s_vv = vv[0] * vv[0] + vv[1] * vv[1] + vv[2] * vv[2] + eps
        dotuv = uv[0] * vv[0] + uv[1] * vv[1] + uv[2] * vv[2]
        cosang = jnp.clip(dotuv * _vrsqrt(s_uv * s_vv),
                          jnp.float32(-1.0 + 1e-6), jnp.float32(1.0 - 1e-6))
        theta = _vacos(cosang)
        b1 = dx
        b2 = vv
        b3 = tuple(x3[c] - x2[c] for c in range(3))

        def cross(u, v):
            return (u[1] * v[2] - u[2] * v[1],
                    u[2] * v[0] - u[0] * v[2],
                    u[0] * v[1] - u[1] * v[0])

        n1 = cross(b1, b2)
        n2 = cross(b2, b3)
        s_b2 = b2[0] * b2[0] + b2[1] * b2[1] + b2[2] * b2[2]
        inv_b2 = jnp.float32(1.0) / (_vsqrt(s_b2) + eps)
        m1 = cross(n1, tuple(b2[c] * inv_b2 for c in range(3)))
        y = m1[0] * n2[0] + m1[1] * n2[1] + m1[2] * n2[2]
        x = n1[0] * n2[0] + n1[1] * n2[1] + n1[2] * n2[2] + eps
        den = x * x + y * y + jnp.float32(1e-30)
        cos2phi = (x * x - y * y) / den
        sin2phi = jnp.float32(2.0) * x * y / den

        dl = d01 - l0
        da = theta - t0
        E = (k_len * dl * dl + k_ang * da * da
             + k_tor * (jnp.float32(1.0) + cos2phi * cp0 + sin2phi * sp0))
        return acc + E * maskf

    acc = lax.fori_loop(0, (B * S) // L, intra_body,
                        jnp.zeros((L,), jnp.float32))

    def inter_body(it, acc):
        e = it * L + lanes
        b = e >> 1
        j = e & 1
        t1 = _gather(bt_v, b)
        ci = b * 4 + j * 2
        b2i = _gather(conns_v, ci)
        c2 = _gather(conns_v, ci + 1) & 1
        t2 = _gather(bt_v, b2i)
        a1 = _gather(tab_v, 6144 + t1 * 2 + j)
        a2 = _gather(tab_v, 6144 + t2 * 2 + c2)
        g1 = b * 32 + a1
        g2 = b2i * 32 + a2
        d2 = eps
        for cv in (cx_v, cy_v, cz_v):
            dc = _gather(cv, g2) - _gather(cv, g1)
            d2 = d2 + dc * dc
        dd = _vsqrt(d2) - jnp.float32(1.5)
        return acc + jnp.float32(0.5) * dd * dd

    acc = lax.fori_loop(0, (B * 2) // L, inter_body, acc)

    total = jnp.sum(acc)
    res_v[...] = jnp.where(lanes == 0, jnp.broadcast_to(total, (L,)),
                           jnp.float32(0.0))
    pltpu.sync_copy(res_v, out_h.at[wid])


@jax.jit
def _run(cx, cy, cz, bt, conns1, tab, hv0, hv1, hv2, hv3, hv4, hv5):
    mesh = plsc.VectorSubcoreMesh(core_axis_name="c", subcore_axis_name="s")
    f = pl.kernel(
        _body,
        out_type=jax.ShapeDtypeStruct((P, L), jnp.float32),
        mesh=mesh,
        compiler_params=pltpu.CompilerParams(needs_layout_passes=False),
        scratch_types=[
            pltpu.VMEM((N,), jnp.float32),
            pltpu.VMEM((N,), jnp.float32),
            pltpu.VMEM((N,), jnp.float32),
            pltpu.VMEM((B,), jnp.int32),
            pltpu.VMEM((B * 4,), jnp.int32),
            pltpu.VMEM((22624,), jnp.int32),
            pltpu.VMEM((64,), jnp.int32),
            pltpu.VMEM((64,), jnp.int32),
            pltpu.VMEM((64,), jnp.int32),
            pltpu.VMEM((64,), jnp.float32),
            pltpu.VMEM((64,), jnp.float32),
            pltpu.VMEM((64,), jnp.float32),
            pltpu.VMEM((64,), jnp.float32),
            pltpu.VMEM((64,), jnp.float32),
            pltpu.VMEM((64,), jnp.float32),
            pltpu.VMEM((64,), jnp.float32),
            pltpu.VMEM((64,), jnp.float32),
            pltpu.VMEM((64,), jnp.float32),
            pltpu.VMEM((64,), jnp.float32),
            pltpu.VMEM((64,), jnp.float32),
            pltpu.VMEM((64,), jnp.float32),
            pltpu.VMEM((512,), jnp.float32),
            pltpu.VMEM_SHARED((T * S * 8,), jnp.float32),
            pltpu.VMEM((T * S * 8,), jnp.float32),
            pltpu.VMEM((L,), jnp.float32),
            pltpu.SemaphoreType.DMA,
        ],
    )
    return f(cx, cy, cz, bt, conns1, tab, hv0, hv1, hv2, hv3, hv4, hv5)


def kernel(coords, pose_stack_block_coord_offset, pose_stack_block_types,
           pose_stack_inter_block_connections, atom_paths_from_conn,
           atom_unique_ids, atom_wildcard_ids, hash_keys, hash_values,
           cart_subgraphs, cart_subgraph_offsets, max_subgraphs_per_block):
    cx = coords[:, :, 0]
    cy = coords[:, :, 1]
    cz = coords[:, :, 2]
    conns1 = pose_stack_inter_block_connections.reshape(P, B * 4)
    tab = jnp.concatenate([
        cart_subgraphs.transpose(2, 0, 1).reshape(4 * T * S),
        atom_unique_ids.reshape(T * A),
        atom_wildcard_ids.reshape(T * A),
        atom_paths_from_conn[:, :, 0].reshape(T * 2),
        cart_subgraph_offsets,
        hash_keys,
    ])
    hv = [hash_values[:, c] for c in range(6)]
    out = _run(cx, cy, cz, pose_stack_block_types, conns1, tab, *hv)
    return out[:, 0]
